# baseline (speedup 1.0000x reference)
import jax
import jax.numpy as jnp
from jax import lax
from jax.experimental import pallas as pl

K_NN = 32


def _body(xf_ref, w1_ref, w2_ref, out_ref):
    N = xf_ref.shape[1]
    DC = xf_ref.shape[2]
    C = w1_ref.shape[0]
    D = DC // C

    xf = xf_ref[0]

    w1 = w1_ref[...]
    w1n = w1 / jnp.maximum(
        jnp.sqrt(jnp.sum(w1 * w1, axis=1, keepdims=True)), 1e-12)
    wt = jnp.concatenate([w1n.T] * D, axis=1)
    w2 = w2_ref[...]
    w2n = w2 / jnp.maximum(
        jnp.sqrt(jnp.sum(w2 * w2, axis=0, keepdims=True)), 1e-12)

    sq = jnp.sum(xf * xf, axis=1, keepdims=True)
    inner = lax.dot_general(xf, xf, (((1,), (1,)), ((), ())),
                            preferred_element_type=jnp.float32)
    adj = sq - 2.0 * inner + sq.T

    iota = lax.broadcasted_iota(jnp.int32, (N, N), 1)
    big = jnp.int32(1 << 30)
    kiota = lax.broadcasted_iota(jnp.int32, (K_NN, DC), 0)
    inf = jnp.float32(jnp.inf)
    xfext = jnp.concatenate([xf, jnp.ones((N, 1), jnp.float32)], axis=1)

    def gdot(ohf):
        return lax.dot_general(ohf, xfext, (((1,), (0,)), ((), ())),
                               preferred_element_type=jnp.float32)

    def step(k, carry):
        adj, rowmin, acc = carry
        tied = adj == rowmin
        gext = gdot(tied.astype(jnp.float32))
        cnt = gext[:, DC:DC + 1]
        anyties = jnp.max(cnt) > 1.5

        def fix(_):
            idxm = jnp.min(jnp.where(tied, iota, big), axis=1, keepdims=True)
            oh2 = iota == idxm
            return gdot(oh2.astype(jnp.float32)), jnp.where(oh2, inf, adj)

        def happy(_):
            return gext, jnp.where(tied, inf, adj)

        gextf, adjm = lax.cond(anyties, fix, happy, None)
        wk = jnp.sum(jnp.where(kiota == k, wt, 0.0), axis=0, keepdims=True)
        acc = acc + gextf[:, :DC] * wk
        rowmin2 = jnp.min(adjm, axis=1, keepdims=True)
        return adjm, rowmin2, acc

    acc0 = jnp.zeros((N, DC), dtype=jnp.float32)
    rowmin0 = jnp.min(adj, axis=1, keepdims=True)
    _, _, acc = lax.fori_loop(0, K_NN, step, (adj, rowmin0, acc0))

    pieces = []
    for d in range(D):
        pieces.append(lax.dot_general(acc[:, d * C:(d + 1) * C], w2n,
                                      (((1,), (0,)), ((), ())),
                                      preferred_element_type=jnp.float32))
    out_ref[0] = jnp.concatenate(pieces, axis=1)


def kernel(x, w1, w2, conv_w, conv_b):
    B, N, D, C = x.shape
    O = w2.shape[1]
    xf = x.reshape(B, N, D * C)
    out = pl.pallas_call(
        _body,
        grid=(B,),
        in_specs=[
            pl.BlockSpec((1, N, D * C), lambda b: (b, 0, 0)),
            pl.BlockSpec((C, K_NN), lambda b: (0, 0)),
            pl.BlockSpec((C, O), lambda b: (0, 0)),
        ],
        out_specs=pl.BlockSpec((1, N, D * O), lambda b: (b, 0, 0)),
        out_shape=jax.ShapeDtypeStruct((B, N, D * O), jnp.float32),
    )(xf, w1, w2)
    return out.reshape(B, N, D, O)


# speedup vs baseline: 1.2758x; 1.2758x over previous
import jax
import jax.numpy as jnp
from jax import lax
from jax.experimental import pallas as pl

K_NN = 32


def _body(xf_ref, w1_ref, w2_ref, out_ref):
    N = xf_ref.shape[1]
    DC = xf_ref.shape[2]
    C = w1_ref.shape[0]
    D = DC // C

    xf = xf_ref[0]

    w1 = w1_ref[...]
    w1n = w1 / jnp.maximum(
        jnp.sqrt(jnp.sum(w1 * w1, axis=1, keepdims=True)), 1e-12)
    wt = jnp.concatenate([w1n.T] * D, axis=1)
    w2 = w2_ref[...]
    w2n = w2 / jnp.maximum(
        jnp.sqrt(jnp.sum(w2 * w2, axis=0, keepdims=True)), 1e-12)

    sq = jnp.sum(xf * xf, axis=1, keepdims=True)
    inner = lax.dot_general(xf, xf, (((1,), (1,)), ((), ())),
                            preferred_element_type=jnp.float32)
    adj = sq - 2.0 * inner + sq.T

    big = jnp.int32(1 << 30)
    kiota = lax.broadcasted_iota(jnp.int32, (K_NN, DC), 0)
    inf = jnp.float32(jnp.inf)

    def step(k, carry):
        adj, acc = carry
        colmin = jnp.min(adj, axis=0, keepdims=True)
        iota = lax.broadcasted_iota(jnp.int32, (N, N), 0)
        tied = adj == colmin
        key = jnp.where(tied, iota, big)
        idxm = jnp.min(key, axis=0, keepdims=True)
        onehot = iota == idxm
        g = lax.dot_general(onehot.astype(jnp.float32), xf,
                            (((0,), (0,)), ((), ())),
                            preferred_element_type=jnp.float32)
        wk = jnp.sum(jnp.where(kiota == k, wt, 0.0), axis=0, keepdims=True)
        acc = acc + g * wk
        adj = jnp.where(onehot, inf, adj)
        return adj, acc

    acc0 = jnp.zeros((N, DC), dtype=jnp.float32)
    _, acc = lax.fori_loop(0, K_NN, step, (adj, acc0))

    pieces = []
    for d in range(D):
        pieces.append(lax.dot_general(acc[:, d * C:(d + 1) * C], w2n,
                                      (((1,), (0,)), ((), ())),
                                      preferred_element_type=jnp.float32))
    out_ref[0] = jnp.concatenate(pieces, axis=1)


def kernel(x, w1, w2, conv_w, conv_b):
    B, N, D, C = x.shape
    O = w2.shape[1]
    xf = x.reshape(B, N, D * C)
    out = pl.pallas_call(
        _body,
        grid=(B,),
        in_specs=[
            pl.BlockSpec((1, N, D * C), lambda b: (b, 0, 0)),
            pl.BlockSpec((C, K_NN), lambda b: (0, 0)),
            pl.BlockSpec((C, O), lambda b: (0, 0)),
        ],
        out_specs=pl.BlockSpec((1, N, D * O), lambda b: (b, 0, 0)),
        out_shape=jax.ShapeDtypeStruct((B, N, D * O), jnp.float32),
    )(xf, w1, w2)
    return out.reshape(B, N, D, O)


# speedup vs baseline: 1.7927x; 1.4051x over previous
import jax
import jax.numpy as jnp
from jax import lax
from jax.experimental import pallas as pl

K_NN = 32


def _body(xf_ref, w1_ref, w2_ref, out_ref):
    PB = xf_ref.shape[0]
    N = xf_ref.shape[1]
    DC = xf_ref.shape[2]
    C = w1_ref.shape[0]
    D = DC // C

    w1 = w1_ref[...]
    w1n = w1 / jnp.maximum(
        jnp.sqrt(jnp.sum(w1 * w1, axis=1, keepdims=True)), 1e-12)
    wt = jnp.concatenate([w1n.T] * D, axis=1)
    w2 = w2_ref[...]
    w2n = w2 / jnp.maximum(
        jnp.sqrt(jnp.sum(w2 * w2, axis=0, keepdims=True)), 1e-12)

    big = jnp.int32(1 << 30)
    inf = jnp.float32(jnp.inf)
    kiota = lax.broadcasted_iota(jnp.int32, (K_NN, DC), 0)

    xfs = [xf_ref[p] for p in range(PB)]
    adjs = []
    for p in range(PB):
        xf = xfs[p]
        sq = jnp.sum(xf * xf, axis=1, keepdims=True)
        inner = lax.dot_general(xf, xf, (((1,), (1,)), ((), ())),
                                preferred_element_type=jnp.float32)
        adjs.append(sq - 2.0 * inner + sq.T)

    def step(k, carry):
        adjs, accs = carry
        wk = jnp.sum(jnp.where(kiota == k, wt, 0.0), axis=0, keepdims=True)
        new_adjs, new_accs = [], []
        for p in range(PB):
            adj, acc = adjs[p], accs[p]
            iota = lax.broadcasted_iota(jnp.int32, (N, N), 1)
            rowmin = jnp.min(adj, axis=1, keepdims=True)
            tied = adj == rowmin
            idxm = jnp.min(jnp.where(tied, iota, big), axis=1, keepdims=True)
            onehot = iota == idxm
            g = lax.dot_general(onehot.astype(jnp.float32), xfs[p],
                                (((1,), (0,)), ((), ())),
                                preferred_element_type=jnp.float32)
            new_accs.append(acc + g * wk)
            new_adjs.append(jnp.where(onehot, inf, adj))
        return tuple(new_adjs), tuple(new_accs)

    acc0 = tuple(jnp.zeros((N, DC), dtype=jnp.float32) for _ in range(PB))
    _, accs = lax.fori_loop(0, K_NN, step, (tuple(adjs), acc0))

    for p in range(PB):
        pieces = []
        for d in range(D):
            pieces.append(lax.dot_general(accs[p][:, d * C:(d + 1) * C], w2n,
                                          (((1,), (0,)), ((), ())),
                                          preferred_element_type=jnp.float32))
        out_ref[p] = jnp.concatenate(pieces, axis=1)


def kernel(x, w1, w2, conv_w, conv_b):
    B, N, D, C = x.shape
    O = w2.shape[1]
    PB = 2
    xf = x.reshape(B, N, D * C)
    out = pl.pallas_call(
        _body,
        grid=(B // PB,),
        in_specs=[
            pl.BlockSpec((PB, N, D * C), lambda b: (b, 0, 0)),
            pl.BlockSpec((C, K_NN), lambda b: (0, 0)),
            pl.BlockSpec((C, O), lambda b: (0, 0)),
        ],
        out_specs=pl.BlockSpec((PB, N, D * O), lambda b: (b, 0, 0)),
        out_shape=jax.ShapeDtypeStruct((B, N, D * O), jnp.float32),
    )(xf, w1, w2)
    return out.reshape(B, N, D, O)


# speedup vs baseline: 1.9284x; 1.0757x over previous
import jax
import jax.numpy as jnp
from jax import lax
from jax.experimental import pallas as pl

K_NN = 32


def _body(xf_ref, w1_ref, w2_ref, out_ref):
    PB = xf_ref.shape[0]
    N = xf_ref.shape[1]
    DC = xf_ref.shape[2]
    C = w1_ref.shape[0]
    D = DC // C

    w1 = w1_ref[...]
    w1n = w1 / jnp.maximum(
        jnp.sqrt(jnp.sum(w1 * w1, axis=1, keepdims=True)), 1e-12)
    wt = jnp.concatenate([w1n.T] * D, axis=1)
    w2 = w2_ref[...]
    w2n = w2 / jnp.maximum(
        jnp.sqrt(jnp.sum(w2 * w2, axis=0, keepdims=True)), 1e-12)

    big = jnp.int32(1 << 30)
    inf = jnp.float32(jnp.inf)
    kiota = lax.broadcasted_iota(jnp.int32, (K_NN, DC), 0)

    xfs = [xf_ref[p] for p in range(PB)]
    adjs = []
    for p in range(PB):
        xf = xfs[p]
        sq = jnp.sum(xf * xf, axis=1, keepdims=True)
        inner = lax.dot_general(xf, xf, (((1,), (1,)), ((), ())),
                                preferred_element_type=jnp.float32)
        adjs.append(sq - 2.0 * inner + sq.T)

    def step(k, carry):
        adjs, accs = carry
        wk = jnp.sum(jnp.where(kiota == k, wt, 0.0), axis=0, keepdims=True)
        new_adjs, new_accs = [], []
        for p in range(PB):
            adj, acc = adjs[p], accs[p]
            iota = lax.broadcasted_iota(jnp.int32, (N, N), 1)
            rowmin = jnp.min(adj, axis=1, keepdims=True)
            tied = adj == rowmin
            idxm = jnp.min(jnp.where(tied, iota, big), axis=1, keepdims=True)
            onehot = iota == idxm
            g = lax.dot_general(onehot.astype(jnp.float32), xfs[p],
                                (((1,), (0,)), ((), ())),
                                preferred_element_type=jnp.float32)
            new_accs.append(acc + g * wk)
            new_adjs.append(jnp.where(onehot, inf, adj))
        return tuple(new_adjs), tuple(new_accs)

    acc0 = tuple(jnp.zeros((N, DC), dtype=jnp.float32) for _ in range(PB))
    _, accs = lax.fori_loop(0, K_NN, step, (tuple(adjs), acc0))

    for p in range(PB):
        pieces = []
        for d in range(D):
            pieces.append(lax.dot_general(accs[p][:, d * C:(d + 1) * C], w2n,
                                          (((1,), (0,)), ((), ())),
                                          preferred_element_type=jnp.float32))
        out_ref[p] = jnp.concatenate(pieces, axis=1)


def kernel(x, w1, w2, conv_w, conv_b):
    B, N, D, C = x.shape
    O = w2.shape[1]
    PB = 4
    xf = x.reshape(B, N, D * C)
    out = pl.pallas_call(
        _body,
        grid=(B // PB,),
        in_specs=[
            pl.BlockSpec((PB, N, D * C), lambda b: (b, 0, 0)),
            pl.BlockSpec((C, K_NN), lambda b: (0, 0)),
            pl.BlockSpec((C, O), lambda b: (0, 0)),
        ],
        out_specs=pl.BlockSpec((PB, N, D * O), lambda b: (b, 0, 0)),
        out_shape=jax.ShapeDtypeStruct((B, N, D * O), jnp.float32),
    )(xf, w1, w2)
    return out.reshape(B, N, D, O)
